# Initial kernel scaffold; baseline (speedup 1.0000x reference)
import jax
import jax.numpy as jnp
from jax.experimental import pallas as pl


def kernel(a, b, a_table, b_table, W_router, b_router, W_in, W_state, b_tile, W_out, W_sum, b_sum, W_diff, b_diff):
    raise NotImplementedError("write your pallas kernel here")


# speedup vs baseline: 1.0391x; 1.0391x over previous
import functools

import jax
import jax.numpy as jnp
from jax import lax
from jax.experimental import pallas as pl
from jax.experimental.pallas import tpu as pltpu
from jax.experimental.pallas import tpu_sc as plsc

_D_MODEL = 1024
_D_HALF = 512
_D_STATE = 256
_N_TILES = 16
_TS = _N_TILES * _D_STATE
_OUT_RANGE = 2047
_B = 4096
_INV_TEMP = 2.0

_CHUNK = 256


def _sc_gather(a_idx, b_idx, a_table, b_table):
    info = plsc.get_sparse_core_info()
    nc, ns = info.num_cores, info.num_subcores
    nw = nc * ns
    bpw = _B // nw

    mesh = plsc.VectorSubcoreMesh(core_axis_name="c", subcore_axis_name="s")

    @functools.partial(
        pl.kernel,
        mesh=mesh,
        out_type=(
            jax.ShapeDtypeStruct((_B, _D_HALF), jnp.float32),
            jax.ShapeDtypeStruct((_B, _D_HALF), jnp.float32),
        ),
        scratch_types=[
            pltpu.VMEM((bpw,), jnp.int32),
            pltpu.VMEM((bpw, _D_HALF), jnp.float32),
            pltpu.SemaphoreType.DMA,
        ],
    )
    def gather_k(a_hbm, b_hbm, at_hbm, bt_hbm, a_out, b_out, idx_v, rows_v, sem):
        wid = lax.axis_index("s") * nc + lax.axis_index("c")
        base = wid * bpw
        pltpu.sync_copy(a_hbm.at[pl.ds(base, bpw)], idx_v)
        pltpu.async_copy(at_hbm.at[idx_v], rows_v, sem).wait()
        pltpu.sync_copy(rows_v, a_out.at[pl.ds(base, bpw)])
        pltpu.sync_copy(b_hbm.at[pl.ds(base, bpw)], idx_v)
        pltpu.async_copy(bt_hbm.at[idx_v], rows_v, sem).wait()
        pltpu.sync_copy(rows_v, b_out.at[pl.ds(base, bpw)])

    return gather_k(a_idx, b_idx, a_table, b_table)


def _tc_body(xa, xb, wr, br, win, btile, eexp, wout, wsum, bsum, wdiff, bdiff,
             probs_o, sum_o, diff_o):
    x = jnp.concatenate([xa[...], xb[...]], axis=1)
    logits = jnp.dot(x, wr[...], preferred_element_type=jnp.float32) + br[...]
    logits = logits * _INV_TEMP
    m = jnp.max(logits, axis=1, keepdims=True)
    e = jnp.exp(logits - m)
    p = e / jnp.sum(e, axis=1, keepdims=True)
    probs_o[...] = p
    xbf = x.astype(jnp.bfloat16)
    h = jnp.tanh(jnp.dot(xbf, win[...], preferred_element_type=jnp.float32)
                 + btile[...])
    pw = jnp.dot(p, eexp[...], preferred_element_type=jnp.float32)
    hw = (h * pw).astype(jnp.bfloat16)
    out = jnp.dot(hw, wout[...],
                  preferred_element_type=jnp.float32).astype(jnp.bfloat16)
    sum_o[...] = jnp.dot(out, wsum[...],
                         preferred_element_type=jnp.float32) + bsum[...]
    diff_o[...] = jnp.dot(out, wdiff[...],
                          preferred_element_type=jnp.float32) + bdiff[...]


def kernel(a, b, a_table, b_table, W_router, b_router, W_in, W_state, b_tile,
           W_out, W_sum, b_sum, W_diff, b_diff):
    a_emb, b_emb = _sc_gather(a.astype(jnp.int32), b.astype(jnp.int32),
                              a_table, b_table)

    win_flat = W_in.transpose(1, 0, 2).reshape(_D_MODEL, _TS).astype(jnp.bfloat16)
    wout_flat = W_out.reshape(_TS, _D_MODEL).astype(jnp.bfloat16)
    wsum_bf = W_sum.astype(jnp.bfloat16)
    wdiff_bf = W_diff.astype(jnp.bfloat16)
    btile_flat = b_tile.reshape(1, _TS)
    br2 = b_router.reshape(1, _N_TILES)
    bsum2 = b_sum.reshape(1, _OUT_RANGE)
    bdiff2 = b_diff.reshape(1, _OUT_RANGE)
    eexp = jnp.repeat(jnp.eye(_N_TILES, dtype=jnp.float32), _D_STATE, axis=1)

    grid = _B // _CHUNK
    row_blk = lambda i: (i, 0)
    full_blk = lambda i: (0, 0)

    probs, sum_logits, diff_logits = pl.pallas_call(
        _tc_body,
        grid=(grid,),
        in_specs=[
            pl.BlockSpec((_CHUNK, _D_HALF), row_blk),
            pl.BlockSpec((_CHUNK, _D_HALF), row_blk),
            pl.BlockSpec((_D_MODEL, _N_TILES), full_blk),
            pl.BlockSpec((1, _N_TILES), full_blk),
            pl.BlockSpec((_D_MODEL, _TS), full_blk),
            pl.BlockSpec((1, _TS), full_blk),
            pl.BlockSpec((_N_TILES, _TS), full_blk),
            pl.BlockSpec((_TS, _D_MODEL), full_blk),
            pl.BlockSpec((_D_MODEL, _OUT_RANGE), full_blk),
            pl.BlockSpec((1, _OUT_RANGE), full_blk),
            pl.BlockSpec((_D_MODEL, _OUT_RANGE), full_blk),
            pl.BlockSpec((1, _OUT_RANGE), full_blk),
        ],
        out_specs=[
            pl.BlockSpec((_CHUNK, _N_TILES), row_blk),
            pl.BlockSpec((_CHUNK, _OUT_RANGE), row_blk),
            pl.BlockSpec((_CHUNK, _OUT_RANGE), row_blk),
        ],
        out_shape=[
            jax.ShapeDtypeStruct((_B, _N_TILES), jnp.float32),
            jax.ShapeDtypeStruct((_B, _OUT_RANGE), jnp.float32),
            jax.ShapeDtypeStruct((_B, _OUT_RANGE), jnp.float32),
        ],
    )(a_emb, b_emb, W_router, br2, win_flat, btile_flat, eexp, wout_flat,
      wsum_bf, bsum2, wdiff_bf, bdiff2)

    return (sum_logits, diff_logits, probs)


# speedup vs baseline: 1.1196x; 1.0775x over previous
import functools

import jax
import jax.numpy as jnp
from jax import lax
from jax.experimental import pallas as pl
from jax.experimental.pallas import tpu as pltpu
from jax.experimental.pallas import tpu_sc as plsc

_D_MODEL = 1024
_D_HALF = 512
_D_STATE = 256
_N_TILES = 16
_TS = _N_TILES * _D_STATE
_OUT_RANGE = 2047
_B = 4096
_INV_TEMP = 2.0

_CHUNK = 512


def _sc_gather(a_idx, b_idx, a_table, b_table):
    info = plsc.get_sparse_core_info()
    nc, ns = info.num_cores, info.num_subcores
    nw = nc * ns
    bpw = _B // nw

    mesh = plsc.VectorSubcoreMesh(core_axis_name="c", subcore_axis_name="s")

    @functools.partial(
        pl.kernel,
        mesh=mesh,
        out_type=(
            jax.ShapeDtypeStruct((_B, _D_HALF), jnp.float32),
            jax.ShapeDtypeStruct((_B, _D_HALF), jnp.float32),
        ),
        scratch_types=[
            pltpu.VMEM((bpw,), jnp.int32),
            pltpu.VMEM((bpw, _D_HALF), jnp.float32),
            pltpu.SemaphoreType.DMA,
        ],
    )
    def gather_k(a_hbm, b_hbm, at_hbm, bt_hbm, a_out, b_out, idx_v, rows_v, sem):
        wid = lax.axis_index("s") * nc + lax.axis_index("c")
        base = wid * bpw
        pltpu.sync_copy(a_hbm.at[pl.ds(base, bpw)], idx_v)
        pltpu.async_copy(at_hbm.at[idx_v], rows_v, sem).wait()
        pltpu.sync_copy(rows_v, a_out.at[pl.ds(base, bpw)])
        pltpu.sync_copy(b_hbm.at[pl.ds(base, bpw)], idx_v)
        pltpu.async_copy(bt_hbm.at[idx_v], rows_v, sem).wait()
        pltpu.sync_copy(rows_v, b_out.at[pl.ds(base, bpw)])

    return gather_k(a_idx, b_idx, a_table, b_table)


def _tc_body(xa, xb, wr, br, win, btile, wout, wsum, bsum, wdiff, bdiff,
             probs_o, sum_o, diff_o):
    x = jnp.concatenate([xa[...], xb[...]], axis=1)
    logits = jnp.dot(x, wr[...], preferred_element_type=jnp.float32) + br[...]
    logits = logits * _INV_TEMP
    m = jnp.max(logits, axis=1, keepdims=True)
    e = jnp.exp(logits - m)
    p = e / jnp.sum(e, axis=1, keepdims=True)
    probs_o[...] = p
    xbf = x.astype(jnp.bfloat16)
    out = jnp.zeros((x.shape[0], _D_MODEL), dtype=jnp.float32)
    for t in range(_N_TILES):
        h_t = jnp.tanh(
            jnp.dot(xbf, win[t], preferred_element_type=jnp.float32)
            + btile[:, pl.ds(t * _D_STATE, _D_STATE)])
        hw_t = (h_t * p[:, t:t + 1]).astype(jnp.bfloat16)
        out = out + jnp.dot(hw_t, wout[t], preferred_element_type=jnp.float32)
    out = out.astype(jnp.bfloat16)
    sum_o[...] = jnp.dot(out, wsum[...],
                         preferred_element_type=jnp.float32) + bsum[...]
    diff_o[...] = jnp.dot(out, wdiff[...],
                          preferred_element_type=jnp.float32) + bdiff[...]


def kernel(a, b, a_table, b_table, W_router, b_router, W_in, W_state, b_tile,
           W_out, W_sum, b_sum, W_diff, b_diff):
    a_emb, b_emb = _sc_gather(a.astype(jnp.int32), b.astype(jnp.int32),
                              a_table, b_table)

    win_bf = W_in.astype(jnp.bfloat16)
    wout_bf = W_out.astype(jnp.bfloat16)
    wsum_bf = W_sum.astype(jnp.bfloat16)
    wdiff_bf = W_diff.astype(jnp.bfloat16)
    btile_flat = b_tile.reshape(1, _TS)
    br2 = b_router.reshape(1, _N_TILES)
    bsum2 = b_sum.reshape(1, _OUT_RANGE)
    bdiff2 = b_diff.reshape(1, _OUT_RANGE)

    grid = _B // _CHUNK
    row_blk = lambda i: (i, 0)
    full_blk = lambda i: (0, 0)

    probs, sum_logits, diff_logits = pl.pallas_call(
        _tc_body,
        grid=(grid,),
        in_specs=[
            pl.BlockSpec((_CHUNK, _D_HALF), row_blk),
            pl.BlockSpec((_CHUNK, _D_HALF), row_blk),
            pl.BlockSpec((_D_MODEL, _N_TILES), full_blk),
            pl.BlockSpec((1, _N_TILES), full_blk),
            pl.BlockSpec((_N_TILES, _D_MODEL, _D_STATE), lambda i: (0, 0, 0)),
            pl.BlockSpec((1, _TS), full_blk),
            pl.BlockSpec((_N_TILES, _D_STATE, _D_MODEL), lambda i: (0, 0, 0)),
            pl.BlockSpec((_D_MODEL, _OUT_RANGE), full_blk),
            pl.BlockSpec((1, _OUT_RANGE), full_blk),
            pl.BlockSpec((_D_MODEL, _OUT_RANGE), full_blk),
            pl.BlockSpec((1, _OUT_RANGE), full_blk),
        ],
        out_specs=[
            pl.BlockSpec((_CHUNK, _N_TILES), row_blk),
            pl.BlockSpec((_CHUNK, _OUT_RANGE), row_blk),
            pl.BlockSpec((_CHUNK, _OUT_RANGE), row_blk),
        ],
        out_shape=[
            jax.ShapeDtypeStruct((_B, _N_TILES), jnp.float32),
            jax.ShapeDtypeStruct((_B, _OUT_RANGE), jnp.float32),
            jax.ShapeDtypeStruct((_B, _OUT_RANGE), jnp.float32),
        ],
    )(a_emb, b_emb, W_router, br2, win_bf, btile_flat, wout_bf,
      wsum_bf, bsum2, wdiff_bf, bdiff2)

    return (sum_logits, diff_logits, probs)
